# Initial kernel scaffold; baseline (speedup 1.0000x reference)
import jax
import jax.numpy as jnp
from jax.experimental import pallas as pl


def kernel(g2m_efeat, grid_feat, mesh_feat, edge_index, e_W1, e_b1, e_W2, e_b2, e_g, e_beta, s_W1, s_b1, s_W2, s_b2, s_g, s_beta, d_W1, d_b1, d_W2, d_b2, d_g, d_beta):
    raise NotImplementedError("write your pallas kernel here")


# speedup vs baseline: 3.0739x; 3.0739x over previous
import functools

import jax
import jax.numpy as jnp
from jax import lax
from jax.experimental import pallas as pl
from jax.experimental.pallas import tpu as pltpu
from jax.experimental.pallas import tpu_sc as plsc

H = 128
N_GRID = 10000
N_MESH = 10000
E = 320000
CH = 128
NCH = E // CH
NC, NS = 2, 16
NW = NC * NS
NB = 1000
EB = 2000


def _ln(h, g, b):
    mu = jnp.mean(h, axis=-1, keepdims=True)
    var = jnp.mean((h - mu) ** 2, axis=-1, keepdims=True)
    return (h - mu) * lax.rsqrt(var + 1e-5) * g + b


def _silu(x):
    return x * jax.nn.sigmoid(x)


def _dot(a, b):
    return jnp.dot(a, b, preferred_element_type=jnp.float32)


def _prep_body(g_ref, m_ref, w1g_ref, w1m_ref, b1_ref,
               sw1_ref, sb1_ref, sw2_ref, sb2_ref, sg_ref, sbt_ref,
               gp_ref, mp_ref, gn_ref):
    g = g_ref[...]
    m = m_ref[...]
    gp_ref[...] = _dot(g, w1g_ref[...]) + b1_ref[...]
    mp_ref[...] = _dot(m, w1m_ref[...])
    h = _silu(_dot(g, sw1_ref[...]) + sb1_ref[...])
    h2 = _dot(h, sw2_ref[...]) + sb2_ref[...]
    gn_ref[...] = g + _ln(h2, sg_ref[...], sbt_ref[...])


def _node_spec():
    return pl.BlockSpec((NB, H), lambda i: (i, 0))


def _full_spec(shape):
    return pl.BlockSpec(shape, lambda i: tuple(0 for _ in shape))


_prep_call = pl.pallas_call(
    _prep_body,
    grid=(N_GRID // NB,),
    in_specs=[_node_spec(), _node_spec(),
              _full_spec((H, H)), _full_spec((H, H)), _full_spec((1, H)),
              _full_spec((H, H)), _full_spec((1, H)), _full_spec((H, H)),
              _full_spec((1, H)), _full_spec((1, H)), _full_spec((1, H))],
    out_specs=[_node_spec(), _node_spec(), _node_spec()],
    out_shape=[jax.ShapeDtypeStruct((N_GRID, H), jnp.float32)] * 3,
)


_mesh = plsc.VectorSubcoreMesh(core_axis_name="c", subcore_axis_name="s",
                               num_cores=NC, num_subcores=NS)

_K_GATHER = -(-NCH // NW)


@functools.partial(
    pl.kernel,
    out_type=(jax.ShapeDtypeStruct((E, H), jnp.float32),
              jax.ShapeDtypeStruct((E, H), jnp.float32)),
    mesh=_mesh,
    scratch_types=[pltpu.VMEM((CH,), jnp.int32), pltpu.VMEM((CH,), jnp.int32),
                   pltpu.VMEM((CH, H), jnp.float32),
                   pltpu.VMEM((CH, H), jnp.float32),
                   pltpu.SemaphoreType.DMA, pltpu.SemaphoreType.DMA])
def _sc_gather(gp_hbm, mp_hbm, src_hbm, dst_hbm, osrc_hbm, odst_hbm,
               idx_a, idx_b, buf_a, buf_b, sem_a, sem_b):
    wid = lax.axis_index("s") * NC + lax.axis_index("c")

    def body(k, carry):
        c = wid + NW * k

        @pl.when(c < NCH)
        def _():
            pltpu.sync_copy(src_hbm.at[pl.ds(c * CH, CH)], idx_a)
            pltpu.sync_copy(dst_hbm.at[pl.ds(c * CH, CH)], idx_b)
            cp_a = pltpu.async_copy(gp_hbm.at[idx_a], buf_a, sem_a)
            cp_b = pltpu.async_copy(mp_hbm.at[idx_b], buf_b, sem_b)
            cp_a.wait()
            cp_b.wait()
            pltpu.sync_copy(buf_a, osrc_hbm.at[pl.ds(c * CH, CH)])
            pltpu.sync_copy(buf_b, odst_hbm.at[pl.ds(c * CH, CH)])

        return carry

    lax.fori_loop(0, _K_GATHER, body, 0)


def _edge_body(x_ref, a_ref, b_ref, w1_ref, w2_ref, b2_ref, g_ref, bt_ref,
               o_ref):
    x = _dot(x_ref[...], w1_ref[...]) + a_ref[...] + b_ref[...]
    h2 = _dot(_silu(x), w2_ref[...]) + b2_ref[...]
    o_ref[...] = _ln(h2, g_ref[...], bt_ref[...])


def _edge_spec():
    return pl.BlockSpec((EB, H), lambda i: (i, 0))


_edge_call = pl.pallas_call(
    _edge_body,
    grid=(E // EB,),
    in_specs=[_edge_spec(), _edge_spec(), _edge_spec(),
              _full_spec((H, H)), _full_spec((H, H)), _full_spec((1, H)),
              _full_spec((1, H)), _full_spec((1, H))],
    out_specs=[_edge_spec()],
    out_shape=[jax.ShapeDtypeStruct((E, H), jnp.float32)],
)


_HALF = NCH // NC
_K_SCAT = -(-_HALF // NS)
_NM_PAD = 10240
_ROWS = _NM_PAD // NS


@functools.partial(
    pl.kernel,
    out_type=jax.ShapeDtypeStruct((NC * _NM_PAD, H), jnp.float32),
    mesh=_mesh,
    scratch_types=[pltpu.VMEM_SHARED((_NM_PAD, H), jnp.float32),
                   pltpu.VMEM((CH,), jnp.int32),
                   pltpu.VMEM((CH, H), jnp.float32)])
def _sc_scatter(ef_hbm, dst_hbm, zero_hbm, out_hbm, acc_shared, idx_v, buf):
    cid = lax.axis_index("c")
    sid = lax.axis_index("s")
    pltpu.sync_copy(zero_hbm.at[pl.ds(sid * _ROWS, _ROWS)],
                    acc_shared.at[pl.ds(sid * _ROWS, _ROWS)])
    plsc.subcore_barrier()

    def body(k, carry):
        j = sid + NS * k
        c = cid * _HALF + j

        @pl.when(j < _HALF)
        def _():
            pltpu.sync_copy(dst_hbm.at[pl.ds(c * CH, CH)], idx_v)
            pltpu.sync_copy(ef_hbm.at[pl.ds(c * CH, CH)], buf)
            pltpu.sync_copy(buf, acc_shared.at[idx_v], add=True)

        return carry

    lax.fori_loop(0, _K_SCAT, body, 0)
    plsc.subcore_barrier()
    pltpu.sync_copy(acc_shared.at[pl.ds(sid * _ROWS, _ROWS)],
                    out_hbm.at[pl.ds(cid * _NM_PAD + sid * _ROWS, _ROWS)])


def _mesh_body(p0_ref, p1_ref, m_ref, w1a_ref, w1b_ref, b1_ref,
               w2_ref, b2_ref, g_ref, bt_ref, o_ref):
    m = m_ref[...]
    agg = p0_ref[...] + p1_ref[...]
    x = _dot(agg, w1a_ref[...]) + _dot(m, w1b_ref[...]) + b1_ref[...]
    h2 = _dot(_silu(x), w2_ref[...]) + b2_ref[...]
    o_ref[...] = m + _ln(h2, g_ref[...], bt_ref[...])


_mesh_call = pl.pallas_call(
    _mesh_body,
    grid=(N_MESH // NB,),
    in_specs=[_node_spec(), _node_spec(), _node_spec(),
              _full_spec((H, H)), _full_spec((H, H)), _full_spec((1, H)),
              _full_spec((H, H)), _full_spec((1, H)), _full_spec((1, H)),
              _full_spec((1, H))],
    out_specs=[_node_spec()],
    out_shape=[jax.ShapeDtypeStruct((N_MESH, H), jnp.float32)],
)


def kernel(g2m_efeat, grid_feat, mesh_feat, edge_index,
           e_W1, e_b1, e_W2, e_b2, e_g, e_beta,
           s_W1, s_b1, s_W2, s_b2, s_g, s_beta,
           d_W1, d_b1, d_W2, d_b2, d_g, d_beta):
    src = edge_index[0]
    dst = edge_index[1]
    r = lambda v: v.reshape(1, H)

    gp, mp, grid_new = _prep_call(
        grid_feat, mesh_feat, e_W1[H:2 * H], e_W1[2 * H:], r(e_b1),
        s_W1, r(s_b1), s_W2, r(s_b2), r(s_g), r(s_beta))

    gsrc, gdst = _sc_gather(gp, mp, src, dst)

    (efeat,) = _edge_call(g2m_efeat, gsrc, gdst, e_W1[:H], e_W2, r(e_b2),
                          r(e_g), r(e_beta))

    parts = _sc_scatter(efeat, dst, jnp.zeros((_NM_PAD, H), jnp.float32))

    (mesh_new,) = _mesh_call(
        parts[:N_MESH], parts[_NM_PAD:_NM_PAD + N_MESH], mesh_feat,
        d_W1[:H], d_W1[H:], r(d_b1), d_W2, r(d_b2), r(d_g), r(d_beta))

    return (grid_new, mesh_new)
